# Initial kernel scaffold; baseline (speedup 1.0000x reference)
import jax
import jax.numpy as jnp
from jax.experimental import pallas as pl


def kernel(pitch, envelope, attention, wavetables):
    raise NotImplementedError("write your pallas kernel here")


# speedup vs baseline: 16.1038x; 16.1038x over previous
import functools

import jax
import jax.numpy as jnp
from jax.experimental import pallas as pl
from jax.experimental.pallas import tpu as pltpu

_SR = 16000
_WT_LEN = 512
_N_WT = 10
_INC_SCALE = _WT_LEN / _SR

_LP = 65536
_C = _LP // 128
_RR = _C // 8


def _synth_body(p_ref, p0_ref, e_ref, att_ref, a_ref, d_ref, o_ref, m_ref):
    inc = p_ref[0] * _INC_SCALE
    inc0 = p0_ref[0] * _INC_SCALE

    lane = jax.lax.broadcasted_iota(jnp.int32, (_C, 128), 1)
    x = inc
    for d in (1, 2, 4, 8, 16, 32, 64):
        x = x + jnp.where(lane >= d, jnp.roll(x, d, axis=1), 0.0)

    rowtot = x[:, 127:128]
    row = jax.lax.broadcasted_iota(jnp.int32, (_C, 1), 0)
    s = rowtot
    for d in (1, 2, 4, 8, 16, 32, 64, 128, 256):
        s = s + jnp.where(row >= d, jnp.roll(s, d, axis=0), 0.0)
    cum = x + (s - rowtot)

    raw = cum - inc0
    m = raw - jnp.floor(raw * (1.0 / _WT_LEN)) * _WT_LEN
    m = jnp.where(_WT_LEN - m < 1e-5, 0.0, m)
    m_ref[...] = m

    a_full = a_ref[...]
    d_full = d_ref[...]

    def chunk(rr, carry):
        base = pl.multiple_of(rr * 8, 8)
        m8 = m_ref[pl.ds(base, 8), :]
        e8 = e_ref[0, pl.ds(base, 8), :]
        att8 = att_ref[0, rr]
        rows = []
        for q in range(8):
            mq = m8[q:q + 1, :]
            lowf = jnp.floor(mq)
            alpha = mq - lowf
            low = jnp.broadcast_to(lowf.astype(jnp.int32), (_N_WT, 128))
            av = jnp.zeros((_N_WT, 128), jnp.float32)
            dv = jnp.zeros((_N_WT, 128), jnp.float32)
            for c in range(4):
                rel = low - (128 * c)
                ok = (rel >= 0) & (rel < 128)
                cl = jnp.clip(rel, 0, 127)
                ga = jnp.take_along_axis(a_full[:, 128 * c:128 * (c + 1)], cl, axis=1)
                gd = jnp.take_along_axis(d_full[:, 128 * c:128 * (c + 1)], cl, axis=1)
                av = jnp.where(ok, ga, av)
                dv = jnp.where(ok, gd, dv)
            val = av + alpha * dv
            rows.append(jnp.sum(val * att8[q], axis=0, keepdims=True))
        o_ref[0, pl.ds(base, 8), :] = jnp.concatenate(rows, axis=0) * e8
        return carry

    jax.lax.fori_loop(0, _RR, chunk, 0)


def kernel(pitch, envelope, attention, wavetables):
    n, l = pitch.shape[0], pitch.shape[1]
    pad = _LP - l

    p2 = jnp.pad(pitch[..., 0], ((0, 0), (0, pad))).reshape(n, _C, 128)
    e2 = jnp.pad(envelope[..., 0], ((0, 0), (0, pad))).reshape(n, _C, 128)
    att5 = jnp.pad(attention, ((0, 0), (0, pad), (0, 0)))
    att5 = att5.reshape(n, _RR, 8, 128, _N_WT).transpose(0, 1, 2, 4, 3)

    a_tab = wavetables
    d_tab = jnp.roll(wavetables, -1, axis=1) - wavetables

    out = pl.pallas_call(
        _synth_body,
        grid=(n,),
        in_specs=[
            pl.BlockSpec((1, _C, 128), lambda i: (i, 0, 0)),
            pl.BlockSpec((1, _C, 128), lambda i: (0, 0, 0)),
            pl.BlockSpec((1, _C, 128), lambda i: (i, 0, 0)),
            pl.BlockSpec((1, _RR, 8, _N_WT, 128), lambda i: (i, 0, 0, 0, 0)),
            pl.BlockSpec((_N_WT, _WT_LEN), lambda i: (0, 0)),
            pl.BlockSpec((_N_WT, _WT_LEN), lambda i: (0, 0)),
        ],
        out_specs=pl.BlockSpec((1, _C, 128), lambda i: (i, 0, 0)),
        out_shape=jax.ShapeDtypeStruct((n, _C, 128), jnp.float32),
        scratch_shapes=[pltpu.VMEM((_C, 128), jnp.float32)],
        compiler_params=pltpu.CompilerParams(
            dimension_semantics=("parallel",)),
    )(p2, p2, e2, att5, a_tab, d_tab)

    return out.reshape(n, _LP)[:, :l, None]
